# Initial kernel scaffold; baseline (speedup 1.0000x reference)
import jax
import jax.numpy as jnp
from jax.experimental import pallas as pl


def kernel(x_spatial, x_temporal, edge_index, edge_weight, W_s1, b_s1, W_s2, b_s2, W_t1, b_t1, W_t2, b_t2, W_gat, att_src, att_dst, b_gat):
    raise NotImplementedError("write your pallas kernel here")


# speedup vs baseline: 17.0097x; 17.0097x over previous
import functools

import jax
import jax.numpy as jnp
from jax import lax
from jax.experimental import pallas as pl
from jax.experimental.pallas import tpu as pltpu
from jax.experimental.pallas import tpu_sc as plsc

N = 10000
E = 320000
D = 128
OUT = 128

NC = 2
NS = 16
L = 16
NW = NC * NS
EPW = E // NW
K = 80
NB = EPW // K
NP = 10240
RPT = NP // NS
DR = NP // L


def _dense_body(xs_ref, xt_ref, ws1, bs1, ws2, bs2, wt1, bt1, wt2, bt2,
                wgs, wgt, asr, adr, h_ref, aa_ref):
    xs = xs_ref[...]
    xt = xt_ref[...]
    s1 = jnp.maximum(jnp.dot(xs, ws1[...], preferred_element_type=jnp.float32)
                     + bs1[...], 0.0)
    s_sc = jnp.dot(s1, ws2[...], preferred_element_type=jnp.float32) + bs2[...]
    t1 = jnp.maximum(jnp.dot(xt, wt1[...], preferred_element_type=jnp.float32)
                     + bt1[...], 0.0)
    t_sc = jnp.dot(t1, wt2[...], preferred_element_type=jnp.float32) + bt2[...]
    m = jnp.maximum(s_sc, t_sc)
    es = jnp.exp(s_sc - m)
    et = jnp.exp(t_sc - m)
    inv = 1.0 / (es + et)
    h = (jnp.dot(xs * (es * inv), wgs[...], preferred_element_type=jnp.float32)
         + jnp.dot(xt * (et * inv), wgt[...], preferred_element_type=jnp.float32))
    h_ref[...] = h
    a_s = jnp.sum(h * asr[...], axis=1, keepdims=True)
    a_d = jnp.sum(h * adr[...], axis=1, keepdims=True)
    aa_ref[...] = jnp.concatenate([a_s, a_d], axis=1)


def _dense_stage(xs, xt, ws1, bs1, ws2, bs2, wt1, bt1, wt2, bt2,
                 wgs, wgt, asr, adr):
    bn = 2000
    grid = (N // bn,)
    full = lambda shape: pl.BlockSpec(shape, lambda i: (0, 0))
    return pl.pallas_call(
        _dense_body,
        grid=grid,
        in_specs=[
            pl.BlockSpec((bn, D), lambda i: (i, 0)),
            pl.BlockSpec((bn, D), lambda i: (i, 0)),
            full((D, 32)), full((1, 32)), full((32, 1)), full((1, 1)),
            full((D, 32)), full((1, 32)), full((32, 1)), full((1, 1)),
            full((D, OUT)), full((D, OUT)),
            full((1, OUT)), full((1, OUT)),
        ],
        out_specs=[
            pl.BlockSpec((bn, OUT), lambda i: (i, 0)),
            pl.BlockSpec((bn, 2), lambda i: (i, 0)),
        ],
        out_shape=[
            jax.ShapeDtypeStruct((N, OUT), jnp.float32),
            jax.ShapeDtypeStruct((N, 2), jnp.float32),
        ],
    )(xs, xt, ws1, bs1, ws2, bs2, wt1, bt1, wt2, bt2, wgs, wgt, asr, adr)


def _sc_body(h_hbm, asrc_hbm, adst_hbm, src_hbm, dst_hbm, out_hbm, den_hbm,
             asv, adv, s1v, d1v, gv, denv, acc, sem):
    cid = lax.axis_index("c")
    sid = lax.axis_index("s")
    wid = sid * NC + cid

    pltpu.sync_copy(asrc_hbm, asv)
    pltpu.sync_copy(adst_hbm, adv)

    z16 = jnp.zeros((L,), jnp.float32)
    iot = lax.iota(jnp.int32, L)

    def zrow(r, _):
        for c in range(D // L):
            gv[r, pl.ds(c * L, L)] = z16
        return 0

    lax.fori_loop(0, K, zrow, 0)
    for i in range(RPT // K):
        pltpu.sync_copy(gv, acc.at[pl.ds(sid * RPT + i * K, K)])

    def zden(r, _):
        denv[r, pl.ds(0, L)] = z16
        return 0

    lax.fori_loop(0, DR, zden, 0)
    plsc.subcore_barrier()

    def batch(b, _):
        pltpu.sync_copy(src_hbm.at[wid, b], s1v.at[0])
        pltpu.sync_copy(dst_hbm.at[wid, b], d1v.at[0])
        pltpu.async_copy(h_hbm.at[s1v.at[0]], gv, sem).wait()
        for j in range(K // L):
            si = s1v[0, pl.ds(j * L, L)]
            di = d1v[0, pl.ds(j * L, L)]
            al = plsc.load_gather(asv, [si]) + plsc.load_gather(adv, [di])
            al = jnp.where(al >= 0.0, al, 0.2 * al)
            p16 = jnp.exp(al)
            for rr in range(L):
                r = j * L + rr
                pr = jnp.full((L,), p16[rr])
                for c in range(D // L):
                    gv[r, pl.ds(c * L, L)] = gv[r, pl.ds(c * L, L)] * pr
                dsc = di[rr]
                plsc.addupdate(denv.at[dsc // L],
                               jnp.where(iot == dsc % L, pr, 0.0))
        pltpu.sync_copy(gv, acc.at[d1v.at[0]], add=True)
        return 0

    lax.fori_loop(0, NB, batch, 0)
    plsc.subcore_barrier()

    base = sid * RPT
    pltpu.sync_copy(acc.at[pl.ds(base, RPT)],
                    out_hbm.at[cid, pl.ds(base, RPT)])
    pltpu.sync_copy(denv, den_hbm.at[wid])


def _sparse_stage(h, asrc, adst, src2, dst2):
    mesh = plsc.VectorSubcoreMesh(core_axis_name="c", subcore_axis_name="s",
                                  num_cores=NC, num_subcores=NS)
    f = pl.kernel(
        _sc_body,
        out_type=[
            jax.ShapeDtypeStruct((NC, NP, D), jnp.float32),
            jax.ShapeDtypeStruct((NW, DR, L), jnp.float32),
        ],
        mesh=mesh,
        scratch_types=[
            pltpu.VMEM((N,), jnp.float32),
            pltpu.VMEM((N,), jnp.float32),
            pltpu.VMEM((1, K), jnp.int32),
            pltpu.VMEM((1, K), jnp.int32),
            pltpu.VMEM((K, D), jnp.float32),
            pltpu.VMEM((DR, L), jnp.float32),
            pltpu.VMEM_SHARED((NP, D), jnp.float32),
            pltpu.SemaphoreType.DMA,
        ],
        compiler_params=pltpu.CompilerParams(needs_layout_passes=False,
                                             use_tc_tiling_on_sc=False),
    )
    return f(h, asrc, adst, src2, dst2)


def _fin_body(p0, p1, den, bg, out_ref):
    acc = p0[0] + p1[0]
    d = jnp.sum(den[...], axis=0)[:, None]
    out_ref[...] = acc / (d + 1e-16) + bg[...]


def _finalize(parts, denflat, b_gat2):
    bn = 2048
    grid = (pl.cdiv(N, bn),)
    return pl.pallas_call(
        _fin_body,
        grid=grid,
        in_specs=[
            pl.BlockSpec((1, bn, OUT), lambda i: (0, i, 0)),
            pl.BlockSpec((1, bn, OUT), lambda i: (1, i, 0)),
            pl.BlockSpec((NW, bn), lambda i: (0, i)),
            pl.BlockSpec((1, OUT), lambda i: (0, 0)),
        ],
        out_specs=pl.BlockSpec((bn, OUT), lambda i: (i, 0)),
        out_shape=jax.ShapeDtypeStruct((N, OUT), jnp.float32),
    )(parts, parts, denflat, b_gat2)


def kernel(x_spatial, x_temporal, edge_index, edge_weight, W_s1, b_s1, W_s2,
           b_s2, W_t1, b_t1, W_t2, b_t2, W_gat, att_src, att_dst, b_gat):
    del edge_weight
    ws1 = W_s1.T
    wt1 = W_t1.T
    wg = W_gat.T
    wgs = wg[:D]
    wgt = wg[D:]
    h, aa = _dense_stage(x_spatial, x_temporal,
                         ws1, b_s1[None, :], W_s2.T, b_s2[None, :],
                         wt1, b_t1[None, :], W_t2.T, b_t2[None, :],
                         wgs, wgt, att_src[None, :], att_dst[None, :])
    src2 = edge_index[0].reshape(NW, NB, K)
    dst2 = edge_index[1].reshape(NW, NB, K)
    parts, denp = _sparse_stage(h, aa[:, 0], aa[:, 1], src2, dst2)
    return _finalize(parts, denp.reshape(NW, NP), b_gat[None, :])
